# baseline (speedup 1.0000x reference)
import functools
import jax
import jax.numpy as jnp
from jax import lax
from jax.experimental import pallas as pl
from jax.experimental.pallas import tpu as pltpu
from jax.experimental.pallas import tpu_sc as plsc

_V = 100000
_B = 1024
_FILL = 0.1 / (_V - 2)
_NC = 2
_NS = 16
_NW = _NC * _NS
_RPW = _B // _NW
_L = 16


def _sc_body(trg_hbm, out_hbm, trgv, tmpl, patterns, sem_rows, sem_pat):
    wid = lax.axis_index("s") * _NC + lax.axis_index("c")
    base = wid * _RPW
    pltpu.sync_copy(trg_hbm.at[pl.ds(base, _RPW)], trgv)

    lane = lax.iota(jnp.int32, _L)
    fillv = jnp.full((_L,), _FILL, jnp.float32)

    def fill_body(i, _):
        tmpl[pl.ds(i * _L, _L)] = fillv
        return 0

    lax.fori_loop(1, _V // _L, fill_body, 0)
    tmpl[pl.ds(0, _L)] = jnp.where(lane == 0, 0.0, fillv)

    chunks = [trgv[pl.ds(c * _L, _L)] for c in range(_RPW // _L)]
    ts = [chunks[r // _L][r % _L] for r in range(_RPW)]

    for r in range(_RPW):
        @pl.when(ts[r] != 0)
        def _(r=r):
            pltpu.async_copy(tmpl, out_hbm.at[base + r], sem_rows)
    for r in range(_RPW):
        @pl.when(ts[r] != 0)
        def _(r=r):
            pltpu.make_async_copy(tmpl, out_hbm.at[base + r], sem_rows).wait()

    npad = ts[0] * 0
    for r in range(_RPW):
        npad = npad + (ts[r] == 0).astype(jnp.int32)

    @pl.when(npad > 0)
    def _():
        zv = jnp.zeros((_L,), jnp.float32)

        def zfill(i, _):
            tmpl[pl.ds(i * _L, _L)] = zv
            return 0

        lax.fori_loop(0, _V // _L, zfill, 0)
        for r in range(_RPW):
            @pl.when(ts[r] == 0)
            def _(r=r):
                pltpu.async_copy(tmpl, out_hbm.at[base + r], sem_rows)
        for r in range(_RPW):
            @pl.when(ts[r] == 0)
            def _(r=r):
                pltpu.make_async_copy(tmpl, out_hbm.at[base + r], sem_rows).wait()

    def _rep_lane(chunk, k):
        m = jnp.where(lane == k, chunk, 0)
        pre = jnp.cumsum(m)
        suf = jnp.flip(jnp.cumsum(jnp.flip(m)))
        return pre + suf - m

    for r in range(_RPW):
        t_rep = _rep_lane(chunks[r // _L], r % _L)
        loc = t_rep - (t_rep // _L) * _L
        pat = jnp.where(lane == loc, 0.9, fillv)
        pat = jnp.where((t_rep < _L) & (lane == 0), 0.0, pat)
        patterns[pl.ds(r * _L, _L)] = pat

    for r in range(_RPW):
        @pl.when(ts[r] != 0)
        def _(r=r):
            wstart = pl.multiple_of((ts[r] // _L) * _L, 8)
            pltpu.async_copy(
                patterns.at[pl.ds(r * _L, _L)],
                out_hbm.at[base + r].at[pl.ds(wstart, _L)],
                sem_pat,
            )
    for r in range(_RPW):
        @pl.when(ts[r] != 0)
        def _(r=r):
            wstart = pl.multiple_of((ts[r] // _L) * _L, 8)
            pltpu.make_async_copy(
                patterns.at[pl.ds(r * _L, _L)],
                out_hbm.at[base + r].at[pl.ds(wstart, _L)],
                sem_pat,
            ).wait()


_mesh = plsc.VectorSubcoreMesh(core_axis_name="c", subcore_axis_name="s")

_sc_call = functools.partial(
    pl.kernel,
    mesh=_mesh,
    compiler_params=pltpu.CompilerParams(needs_layout_passes=False),
    out_type=jax.ShapeDtypeStruct((_B, _V), jnp.float32),
    scratch_types=[
        pltpu.VMEM((_RPW,), jnp.int32),
        pltpu.VMEM((_V,), jnp.float32),
        pltpu.VMEM((_RPW * _L,), jnp.float32),
        pltpu.SemaphoreType.DMA,
        pltpu.SemaphoreType.DMA,
    ],
)(_sc_body)


def kernel(trg_token_ids_batch):
    trg = trg_token_ids_batch.reshape(_B)
    return _sc_call(trg)
